# baseline (speedup 1.0000x reference)
import functools

import jax
import jax.numpy as jnp
from jax import lax
from jax.experimental import pallas as pl
from jax.experimental.pallas import tpu as pltpu
from jax.experimental.pallas import tpu_sc as plsc

BATCH = 16384
DIM = 64
NUM_WORKERS = 32
B_PER_W = BATCH // NUM_WORKERS
CHUNK = 128
NCHUNK = B_PER_W // CHUNK


def _body(user_table, video_table, user_idx, video_idx, out_hbm,
          idx_u, idx_v, pidx_u, pidx_v, rows_u, rows_v, out_v,
          sem_u0, sem_u1, sem_v0, sem_v1):
    wid = lax.axis_index("s") * 2 + lax.axis_index("c")
    base = wid * B_PER_W

    pltpu.sync_copy(user_idx.at[pl.ds(base, B_PER_W)], idx_u)
    pltpu.sync_copy(video_idx.at[pl.ds(base, B_PER_W)], idx_v)

    def make_pidx(i, carry):
        c = i // (CHUNK // 16)
        r = i % (CHUNK // 16)
        s = pl.ds(i * 16, 16)
        d = pl.ds(r * 16, 16)
        pidx_u[c, d] = lax.shift_right_logical(idx_u[s], 1)
        pidx_v[c, d] = lax.shift_right_logical(idx_v[s], 1)
        return carry

    lax.fori_loop(0, B_PER_W // 16, make_pidx, 0)

    sems_u = (sem_u0, sem_u1)
    sems_v = (sem_v0, sem_v1)

    def issue(c):
        buf = c % 2
        cp_u = pltpu.async_copy(
            user_table.at[pidx_u.at[c]], rows_u.at[buf], sems_u[buf])
        cp_v = pltpu.async_copy(
            video_table.at[pidx_v.at[c]], rows_v.at[buf], sems_v[buf])
        return cp_u, cp_v

    lane = lax.iota(jnp.int32, 16)
    inflight = issue(0)

    for c in range(NCHUNK):
        nxt = issue(c + 1) if c + 1 < NCHUNK else None
        cp_u, cp_v = inflight
        cp_u.wait()
        cp_v.wait()
        buf = c % 2
        bvec = jnp.full((16,), buf, jnp.int32)

        def group(g, carry, c=c, bvec=bvec):
            row_idx = g * 16 + lane
            s = pl.ds(c * CHUNK + g * 16, 16)
            half_u = (idx_u[s] & 1) << 6
            half_v = (idx_v[s] & 1) << 6
            acc = jnp.zeros((16,), jnp.float32)
            for j in range(DIM):
                u = plsc.load_gather(rows_u, [bvec, row_idx, half_u + j])
                v = plsc.load_gather(rows_v, [bvec, row_idx, half_v + j])
                acc = acc + u * v
            out_v[s] = acc
            return carry

        lax.fori_loop(0, CHUNK // 16, group, 0)
        inflight = nxt

    pltpu.sync_copy(out_v, out_hbm.at[pl.ds(base, B_PER_W)])


@jax.jit
def kernel(user_idx, video_idx, user_table, video_table):
    n_packed = user_table.shape[0] // 2
    ut2 = user_table.reshape(n_packed, 2 * DIM)
    vt2 = video_table.reshape(n_packed, 2 * DIM)
    mesh = plsc.VectorSubcoreMesh(core_axis_name="c", subcore_axis_name="s")
    k = functools.partial(
        pl.kernel,
        mesh=mesh,
        out_type=jax.ShapeDtypeStruct((BATCH,), jnp.float32),
        scratch_types=[
            pltpu.VMEM((B_PER_W,), jnp.int32),
            pltpu.VMEM((B_PER_W,), jnp.int32),
            pltpu.VMEM((NCHUNK, CHUNK), jnp.int32),
            pltpu.VMEM((NCHUNK, CHUNK), jnp.int32),
            pltpu.VMEM((2, CHUNK, 2 * DIM), jnp.float32),
            pltpu.VMEM((2, CHUNK, 2 * DIM), jnp.float32),
            pltpu.VMEM((B_PER_W,), jnp.float32),
            pltpu.SemaphoreType.DMA,
            pltpu.SemaphoreType.DMA,
            pltpu.SemaphoreType.DMA,
            pltpu.SemaphoreType.DMA,
        ],
        compiler_params=pltpu.CompilerParams(needs_layout_passes=False),
    )(_body)
    return k(ut2, vt2,
             user_idx.astype(jnp.int32), video_idx.astype(jnp.int32))


# speedup vs baseline: 1.5534x; 1.5534x over previous
import functools

import jax
import jax.numpy as jnp
from jax import lax
from jax.experimental import pallas as pl
from jax.experimental.pallas import tpu as pltpu
from jax.experimental.pallas import tpu_sc as plsc

BATCH = 16384
DIM = 64
NUM_WORKERS = 32
B_PER_W = BATCH // NUM_WORKERS


def _body(user_table, video_table, user_idx, video_idx, out_hbm,
          vidx_u, vidx_v, rows_u, rows_v, out_v, sem_u, sem_v):
    wid = lax.axis_index("s") * 2 + lax.axis_index("c")
    base = wid * B_PER_W

    pltpu.sync_copy(user_idx.at[pl.ds(base, B_PER_W)], vidx_u)
    pltpu.sync_copy(video_idx.at[pl.ds(base, B_PER_W)], vidx_v)

    lane = lax.iota(jnp.int32, 16)
    HALF = B_PER_W // 2

    for c in range(2):
        def fire(i, carry, c=c):
            m = lane == (i & 15)
            s = pl.ds(c * HALF + (i & ~15), 16)
            su = jnp.max(jnp.where(m, vidx_u[s], 0))
            sv = jnp.max(jnp.where(m, vidx_v[s], 0))
            pltpu.async_copy(
                user_table.at[pl.ds(su, 1)],
                rows_u.at[pl.ds(i, 1)], sem_u)
            pltpu.async_copy(
                video_table.at[pl.ds(sv, 1)],
                rows_v.at[pl.ds(i, 1)], sem_v)
            return carry

        lax.fori_loop(0, HALF, fire, 0)

        pltpu.make_async_copy(user_table.at[pl.ds(0, HALF)], rows_u, sem_u).wait()
        pltpu.make_async_copy(video_table.at[pl.ds(0, HALF)], rows_v, sem_v).wait()

        def group(g, carry, c=c):
            row_idx = g * 16 + lane
            acc = jnp.zeros((16,), jnp.float32)
            for j in range(DIM):
                col_idx = jnp.full((16,), j, jnp.int32)
                u = plsc.load_gather(rows_u, [row_idx, col_idx])
                v = plsc.load_gather(rows_v, [row_idx, col_idx])
                acc = acc + u * v
            out_v[pl.ds(c * HALF + g * 16, 16)] = acc
            return carry

        lax.fori_loop(0, HALF // 16, group, 0)

    pltpu.sync_copy(out_v, out_hbm.at[pl.ds(base, B_PER_W)])


@jax.jit
def kernel(user_idx, video_idx, user_table, video_table):
    mesh = plsc.VectorSubcoreMesh(core_axis_name="c", subcore_axis_name="s")
    k = functools.partial(
        pl.kernel,
        mesh=mesh,
        out_type=jax.ShapeDtypeStruct((BATCH,), jnp.float32),
        scratch_types=[
            pltpu.VMEM((B_PER_W,), jnp.int32),
            pltpu.VMEM((B_PER_W,), jnp.int32),
            pltpu.VMEM((B_PER_W // 2, DIM), jnp.float32),
            pltpu.VMEM((B_PER_W // 2, DIM), jnp.float32),
            pltpu.VMEM((B_PER_W,), jnp.float32),
            pltpu.SemaphoreType.DMA,
            pltpu.SemaphoreType.DMA,
        ],
        compiler_params=pltpu.CompilerParams(needs_layout_passes=False),
    )(_body)
    return k(user_table, video_table,
             user_idx.astype(jnp.int32), video_idx.astype(jnp.int32))
